# Initial kernel scaffold; baseline (speedup 1.0000x reference)
import jax
import jax.numpy as jnp
from jax.experimental import pallas as pl


def kernel(input_, embed_weight, embedding_buffer, updates, updates_indices, update_to_batch, hyper_weight_pool_indices, token_to_batch_indices):
    raise NotImplementedError("write your pallas kernel here")


# speedup vs baseline: 3.1889x; 3.1889x over previous
import jax
import jax.numpy as jnp
from jax import lax
from jax.experimental import pallas as pl
from jax.experimental.pallas import tpu as pltpu
from jax.experimental.pallas import tpu_sc as plsc

IVS = 100000
POOL_W = 2048
NKEY = 64 * POOL_W
T = 16384
H = 128
U = 512
S = 8
NC, NS, L = 2, 16, 16
NW = NC * NS
TPW = T // NW
CH = 256
CR = CH // 128
UPW = U // NS


def _body(ids_hbm, embed_hbm, buf_hbm, upd_hbm, ui_hbm, utb_hbm, hwpi_hbm,
          tb_hbm, out_hbm,
          hwpi_v, uidx_v, utb_v, ui_v, keys_v, enc_v,
          ids_v, tbv_v, eidx_v, hkey_v, rank_v, mapu_v, mapuc_v,
          rows_v, hrows_v, encb_v, enc_sh, sem):
  cid = lax.axis_index("c")
  sid = lax.axis_index("s")
  wid = sid * NC + cid
  iota = lax.iota(jnp.int32, L)

  pltpu.sync_copy(hwpi_hbm, hwpi_v)
  for r in range(2):
    pltpu.sync_copy(upd_hbm.at[pl.ds(sid * UPW * S + r * 128, 128)],
                    uidx_v.at[r])
  pltpu.sync_copy(utb_hbm, utb_v)
  pltpu.sync_copy(ui_hbm, ui_v)

  for r in range(2):
    pltpu.async_copy(embed_hbm.at[uidx_v.at[r]],
                     rows_v.at[pl.ds(r * 128, 128), :], sem).wait()
  for g in range(UPW // L):
    subcol = []
    submask = []
    for ss in range(S):
      col = iota * S + ss
      iv = plsc.load_gather(uidx_v, [jnp.full((L,), g, jnp.int32), col])
      subcol.append(col)
      submask.append(iv != 0)
    cnt = jnp.zeros((L,), jnp.float32)
    for ss in range(S):
      cnt = cnt + jnp.where(submask[ss], 1.0, 0.0)
    recip = 1.0 / jnp.maximum(cnt, 1.0)

    def enc_col(col, _):
      colv = jnp.full((L,), col, jnp.int32)
      acc = jnp.zeros((L,), jnp.float32)
      for ss in range(S):
        v = plsc.load_gather(rows_v, [g * 128 + subcol[ss], colv])
        acc = acc + jnp.where(submask[ss], v, 0.0)
      plsc.store_scatter(enc_v, [g * L + iota, colv], acc * recip)
      return _

    lax.fori_loop(0, H, enc_col, 0)
  pltpu.sync_copy(enc_v, enc_sh.at[pl.ds(sid * UPW, UPW), :])

  for g in range(U // L):
    utb_g = utb_v[pl.ds(g * L, L)]
    slot = plsc.load_gather(hwpi_v, [utb_g])
    keys_v[pl.ds(g * L, L)] = slot * POOL_W + ui_v[pl.ds(g * L, L)]

  plsc.subcore_barrier()

  neg1 = jnp.full((L,), -1, jnp.int32)
  for chunk in range(TPW // CH):
    tbase = wid * TPW + chunk * CH
    pltpu.sync_copy(ids_hbm.at[pl.ds(tbase, CH)], ids_v)
    pltpu.sync_copy(tb_hbm.at[pl.ds(tbase, CH)], tbv_v)

    for r in range(CR):
      for j in range(128 // L):
        hkey_v[r, pl.ds(j * L, L)] = iota + r * 128 + j * L

    nh = jnp.int32(0)
    for g in range(CH // L):
      ids = ids_v[pl.ds(g * L, L)]
      tb = tbv_v[pl.ds(g * L, L)]
      slot = plsc.load_gather(hwpi_v, [tb])
      ishyp = ids >= IVS
      key = slot * POOL_W + (ids - IVS)
      incl = plsc.cumsum(jnp.where(ishyp, 1, 0))
      rank = nh + incl - 1
      rclip = jnp.clip(rank, 0, CH - 1)
      plsc.store_scatter(hkey_v, [rclip >> 7, rclip & 127], key, mask=ishyp)
      rank_v[pl.ds(g * L, L)] = rank
      eidx_v[g // 8, pl.ds((g % 8) * L, L)] = jnp.where(
          ishyp, g * L + iota, ids)
      nh = nh + jnp.sum(jnp.where(ishyp, 1, 0))

    cps = []
    for r in range(CR):
      cps.append(pltpu.async_copy(embed_hbm.at[eidx_v.at[r]],
                                  rows_v.at[pl.ds(r * 128, 128), :], sem))
      cps.append(pltpu.async_copy(buf_hbm.at[hkey_v.at[r]],
                                  hrows_v.at[pl.ds(r * 128, 128), :], sem))
    for cp in cps:
      cp.wait()

    for bb in range(CH // L):
      mapu_v[pl.ds(bb * L, L)] = neg1
    anyv = jnp.int32(0)
    for bb in range(CH // L):

      @pl.when(bb * L < nh)
      def _find_group():
        hk = hkey_v[bb // 8, pl.ds((bb % 8) * L, L)]

        def find_body(p, u):
          pk = plsc.load_gather(keys_v, [jnp.full((L,), p, jnp.int32)])
          return jnp.where(pk == hk, p, u)

        u = lax.fori_loop(0, U, find_body, neg1)
        mapu_v[pl.ds(bb * L, L)] = u

      mu = mapu_v[pl.ds(bb * L, L)]
      mapuc_v[bb // 8, pl.ds((bb % 8) * L, L)] = jnp.clip(mu, 0, U - 1)
      valid = jnp.logical_and(mu >= 0, bb * L + iota < nh)
      anyv = anyv + jnp.sum(jnp.where(valid, 1, 0))

    @pl.when(anyv > 0)
    def _override():
      for r in range(CR):
        pltpu.async_copy(enc_sh.at[mapuc_v.at[r]],
                         encb_v.at[pl.ds(r * 128, 128), :], sem).wait()
      for bb in range(CH // L):
        mu = mapu_v[pl.ds(bb * L, L)]
        valid = jnp.logical_and(mu >= 0, bb * L + iota < nh)

        @pl.when(jnp.any(valid))
        def _ov_group():
          def ov_col(col, _):
            colv = jnp.full((L,), col, jnp.int32)
            v = plsc.load_gather(encb_v, [bb * L + iota, colv], mask=valid)
            plsc.store_scatter(hrows_v, [bb * L + iota, colv], v, mask=valid)
            return _

          lax.fori_loop(0, H, ov_col, 0)

    for g in range(CH // L):
      ids = ids_v[pl.ds(g * L, L)]
      ishyp = ids >= IVS

      @pl.when(jnp.any(ishyp))
      def _repl_group():
        rank = jnp.clip(rank_v[pl.ds(g * L, L)], 0, CH - 1)

        def rp_col(col, _):
          colv = jnp.full((L,), col, jnp.int32)
          v = plsc.load_gather(hrows_v, [rank, colv], mask=ishyp)
          plsc.store_scatter(rows_v, [g * L + iota, colv], v, mask=ishyp)
          return _

        lax.fori_loop(0, H, rp_col, 0)

    pltpu.sync_copy(rows_v, out_hbm.at[pl.ds(tbase, CH), :])


@jax.jit
def _run(ids, embed, buf2d, upd_flat, ui, utb, hwpi, tb):
  mesh = plsc.VectorSubcoreMesh(core_axis_name="c", subcore_axis_name="s",
                                num_cores=NC, num_subcores=NS)
  scratch = [
      pltpu.VMEM((64,), jnp.int32),
      pltpu.VMEM((2, 128), jnp.int32),
      pltpu.VMEM((U,), jnp.int32),
      pltpu.VMEM((U,), jnp.int32),
      pltpu.VMEM((U,), jnp.int32),
      pltpu.VMEM((UPW, H), jnp.float32),
      pltpu.VMEM((CH,), jnp.int32),
      pltpu.VMEM((CH,), jnp.int32),
      pltpu.VMEM((CR, 128), jnp.int32),
      pltpu.VMEM((CR, 128), jnp.int32),
      pltpu.VMEM((CH,), jnp.int32),
      pltpu.VMEM((CH,), jnp.int32),
      pltpu.VMEM((CR, 128), jnp.int32),
      pltpu.VMEM((CH, H), jnp.float32),
      pltpu.VMEM((CH, H), jnp.float32),
      pltpu.VMEM((CH, H), jnp.float32),
      pltpu.VMEM_SHARED((U, H), jnp.float32),
      pltpu.SemaphoreType.DMA,
  ]
  f = pl.kernel(
      _body,
      out_type=jax.ShapeDtypeStruct((T, H), jnp.float32),
      mesh=mesh,
      scratch_types=scratch,
      compiler_params=pltpu.CompilerParams(needs_layout_passes=False),
  )
  return f(ids, embed, buf2d, upd_flat, ui, utb, hwpi, tb)


def kernel(input_, embed_weight, embedding_buffer, updates, updates_indices,
           update_to_batch, hyper_weight_pool_indices, token_to_batch_indices):
  ids = input_.astype(jnp.int32)
  buf2d = embedding_buffer.reshape(NKEY, H)
  upd_flat = updates.astype(jnp.int32).reshape(U * S)
  return _run(ids, embed_weight, buf2d, upd_flat,
              updates_indices.astype(jnp.int32),
              update_to_batch.astype(jnp.int32),
              hyper_weight_pool_indices.astype(jnp.int32),
              token_to_batch_indices.astype(jnp.int32))


# speedup vs baseline: 3.7066x; 1.1624x over previous
import jax
import jax.numpy as jnp
from jax import lax
from jax.experimental import pallas as pl
from jax.experimental.pallas import tpu as pltpu
from jax.experimental.pallas import tpu_sc as plsc

IVS = 100000
POOL_W = 2048
NKEY = 64 * POOL_W
T = 16384
H = 128
U = 512
S = 8
NC, NS, L = 2, 16, 16
NW = NC * NS
TPW = T // NW
CH = 256
HB = 64
UPW = U // NS


def _body(ids_hbm, embed_hbm, buf_hbm, upd_hbm, ui_hbm, utb_hbm, hwpi_hbm,
          tb_hbm, out_hbm,
          hwpi_v, uidx_v, utb_v, ui_v, keys_v, enc_v,
          ids0_v, ids1_v, tb0_v, tb1_v,
          eidx0_v, eidx1_v, hkey0_v, hkey1_v, hpos0_v, hpos1_v,
          mapw_v, base_a, base_b, hyp0_v, hyp1_v, srow_v,
          enc_sh, sem_e, sem_g0, sem_g1, sem_w, sem_o):
  cid = lax.axis_index("c")
  sid = lax.axis_index("s")
  wid = sid * NC + cid
  iota = lax.iota(jnp.int32, L)
  neg1 = jnp.full((L,), -1, jnp.int32)
  tbase = wid * TPW

  pltpu.sync_copy(hwpi_hbm, hwpi_v)
  for r in range(2):
    pltpu.sync_copy(upd_hbm.at[pl.ds(sid * UPW * S + r * 128, 128)],
                    uidx_v.at[r])
  pltpu.sync_copy(utb_hbm, utb_v)
  pltpu.sync_copy(ui_hbm, ui_v)
  pltpu.sync_copy(ids_hbm.at[pl.ds(tbase, CH)], ids0_v)
  pltpu.sync_copy(ids_hbm.at[pl.ds(tbase + CH, CH)], ids1_v)
  pltpu.sync_copy(tb_hbm.at[pl.ds(tbase, CH)], tb0_v)
  pltpu.sync_copy(tb_hbm.at[pl.ds(tbase + CH, CH)], tb1_v)

  enc_cps = [pltpu.async_copy(embed_hbm.at[uidx_v.at[r]],
                              base_a.at[pl.ds(r * 128, 128), :], sem_e)
             for r in range(2)]

  def pass1(ids_v, tbv_v, eidx_v, hkey_v, hpos_v):
    for r in range(4):
      for j in range(4):
        hkey_v[r, pl.ds(j * L, L)] = iota + r * HB + j * L
    nh = jnp.int32(0)
    for g in range(CH // L):
      ids = ids_v[pl.ds(g * L, L)]
      tb = tbv_v[pl.ds(g * L, L)]
      slot = plsc.load_gather(hwpi_v, [tb])
      ishyp = ids >= IVS
      key = slot * POOL_W + (ids - IVS)
      incl = plsc.cumsum(jnp.where(ishyp, 1, 0))
      rank = nh + incl - 1
      rclip = jnp.clip(rank, 0, CH - 1)
      plsc.store_scatter(hkey_v, [rclip >> 6, rclip & (HB - 1)], key,
                         mask=ishyp)
      plsc.store_scatter(hpos_v, [rclip], g * L + iota, mask=ishyp)
      eidx_v[g // 8, pl.ds((g % 8) * L, L)] = jnp.where(
          ishyp, g * L + iota, ids)
      nh = nh + jnp.sum(jnp.where(ishyp, 1, 0))
    return nh

  nh0 = pass1(ids0_v, tb0_v, eidx0_v, hkey0_v, hpos0_v)
  nh1 = pass1(ids1_v, tb1_v, eidx1_v, hkey1_v, hpos1_v)

  for cp in enc_cps:
    cp.wait()
  for g in range(UPW // L):
    subcol = []
    submask = []
    for ss in range(S):
      col = iota * S + ss
      iv = plsc.load_gather(uidx_v, [jnp.full((L,), g, jnp.int32), col])
      subcol.append(col)
      submask.append(iv != 0)
    cnt = jnp.zeros((L,), jnp.float32)
    for ss in range(S):
      cnt = cnt + jnp.where(submask[ss], 1.0, 0.0)
    recip = 1.0 / jnp.maximum(cnt, 1.0)

    def enc_col(col, _):
      colv = jnp.full((L,), col, jnp.int32)
      acc = jnp.zeros((L,), jnp.float32)
      for ss in range(S):
        v = plsc.load_gather(base_a, [g * 128 + subcol[ss], colv])
        acc = acc + jnp.where(submask[ss], v, 0.0)
      plsc.store_scatter(enc_v, [g * L + iota, colv], acc * recip)
      return _

    lax.fori_loop(0, H, enc_col, 0)
  pltpu.sync_copy(enc_v, enc_sh.at[pl.ds(sid * UPW, UPW), :])

  cps0 = [pltpu.async_copy(embed_hbm.at[eidx0_v.at[r]],
                           base_a.at[pl.ds(r * 128, 128), :], sem_g0)
          for r in range(2)]
  cps0.append(pltpu.async_copy(buf_hbm.at[hkey0_v.at[0]], hyp0_v, sem_g0))
  cps1 = [pltpu.async_copy(embed_hbm.at[eidx1_v.at[r]],
                           base_b.at[pl.ds(r * 128, 128), :], sem_g1)
          for r in range(2)]
  cps1.append(pltpu.async_copy(buf_hbm.at[hkey1_v.at[0]], hyp1_v, sem_g1))

  for g in range(U // L):
    utb_g = utb_v[pl.ds(g * L, L)]
    slot = plsc.load_gather(hwpi_v, [utb_g])
    keys_v[pl.ds(g * L, L)] = slot * POOL_W + ui_v[pl.ds(g * L, L)]

  plsc.subcore_barrier()

  def compute(base_v, hyp_v, hkey_v, hpos_v, nh, cps):
    for cp in cps:
      cp.wait()
    for b in range(CH // HB):
      if b > 0:
        @pl.when(b * HB < nh)
        def _fetch_batch():
          pltpu.async_copy(buf_hbm.at[hkey_v.at[b]], hyp_v, sem_o).wait()
      for gg in range(HB // L):
        bb = b * (HB // L) + gg

        @pl.when(bb * L < nh)
        def _group():
          hk = hkey_v[b, pl.ds(gg * L, L)]
          lanev = bb * L + iota < nh

          def find_body(i, u):
            for d in range(8):
              p = i * 8 + d
              pk = plsc.load_gather(keys_v, [jnp.full((L,), p, jnp.int32)])
              u = jnp.where(pk == hk, p, u)
            return u

          u = lax.fori_loop(0, U // 8, find_body, neg1)
          valid = jnp.logical_and(u >= 0, lanev)

          @pl.when(jnp.any(valid))
          def _override():
            mapw_v[...] = jnp.clip(u, 0, U - 1)
            pltpu.async_copy(enc_sh.at[mapw_v], srow_v, sem_o).wait()

            def ov_col(c4, _):
              for d in range(4):
                colv = jnp.full((L,), c4 * 4 + d, jnp.int32)
                v = plsc.load_gather(srow_v, [iota, colv], mask=valid)
                plsc.store_scatter(hyp_v, [gg * L + iota, colv], v,
                                   mask=valid)
              return _

            lax.fori_loop(0, H // 4, ov_col, 0)

          pos = jnp.clip(hpos_v[pl.ds(bb * L, L)], 0, CH - 1)

          def rp_col(c4, _):
            for d in range(4):
              colv = jnp.full((L,), c4 * 4 + d, jnp.int32)
              v = plsc.load_gather(hyp_v, [gg * L + iota, colv], mask=lanev)
              plsc.store_scatter(base_v, [pos, colv], v, mask=lanev)
            return _

          lax.fori_loop(0, H // 4, rp_col, 0)

  compute(base_a, hyp0_v, hkey0_v, hpos0_v, nh0, cps0)
  w0 = pltpu.async_copy(base_a, out_hbm.at[pl.ds(tbase, CH), :], sem_w)
  compute(base_b, hyp1_v, hkey1_v, hpos1_v, nh1, cps1)
  w0.wait()
  pltpu.sync_copy(base_b, out_hbm.at[pl.ds(tbase + CH, CH), :])


@jax.jit
def _run(ids, embed, buf2d, upd_flat, ui, utb, hwpi, tb):
  mesh = plsc.VectorSubcoreMesh(core_axis_name="c", subcore_axis_name="s",
                                num_cores=NC, num_subcores=NS)
  scratch = [
      pltpu.VMEM((64,), jnp.int32),
      pltpu.VMEM((2, 128), jnp.int32),
      pltpu.VMEM((U,), jnp.int32),
      pltpu.VMEM((U,), jnp.int32),
      pltpu.VMEM((U,), jnp.int32),
      pltpu.VMEM((UPW, H), jnp.float32),
      pltpu.VMEM((CH,), jnp.int32),
      pltpu.VMEM((CH,), jnp.int32),
      pltpu.VMEM((CH,), jnp.int32),
      pltpu.VMEM((CH,), jnp.int32),
      pltpu.VMEM((2, 128), jnp.int32),
      pltpu.VMEM((2, 128), jnp.int32),
      pltpu.VMEM((4, HB), jnp.int32),
      pltpu.VMEM((4, HB), jnp.int32),
      pltpu.VMEM((CH,), jnp.int32),
      pltpu.VMEM((CH,), jnp.int32),
      pltpu.VMEM((L,), jnp.int32),
      pltpu.VMEM((CH, H), jnp.float32),
      pltpu.VMEM((CH, H), jnp.float32),
      pltpu.VMEM((HB, H), jnp.float32),
      pltpu.VMEM((HB, H), jnp.float32),
      pltpu.VMEM((L, H), jnp.float32),
      pltpu.VMEM_SHARED((U, H), jnp.float32),
      pltpu.SemaphoreType.DMA,
      pltpu.SemaphoreType.DMA,
      pltpu.SemaphoreType.DMA,
      pltpu.SemaphoreType.DMA,
      pltpu.SemaphoreType.DMA,
  ]
  f = pl.kernel(
      _body,
      out_type=jax.ShapeDtypeStruct((T, H), jnp.float32),
      mesh=mesh,
      scratch_types=scratch,
      compiler_params=pltpu.CompilerParams(needs_layout_passes=False),
  )
  return f(ids, embed, buf2d, upd_flat, ui, utb, hwpi, tb)


def kernel(input_, embed_weight, embedding_buffer, updates, updates_indices,
           update_to_batch, hyper_weight_pool_indices, token_to_batch_indices):
  ids = input_.astype(jnp.int32)
  buf2d = embedding_buffer.reshape(NKEY, H)
  upd_flat = updates.astype(jnp.int32).reshape(U * S)
  return _run(ids, embed_weight, buf2d, upd_flat,
              updates_indices.astype(jnp.int32),
              update_to_batch.astype(jnp.int32),
              hyper_weight_pool_indices.astype(jnp.int32),
              token_to_batch_indices.astype(jnp.int32))
